# baseline (speedup 1.0000x reference)
import dataclasses

import jax
import jax.numpy as jnp
from jax import lax
from jax.experimental import pallas as pl
from jax.experimental.pallas import tpu as pltpu
from jax.experimental.pallas import tpu_sc as plsc

N = 10000
NP = 10240
E = 160000
EP = 163840
C = 256
CH = 128
NT = 16
K = 128
GCH = (EP // NT) // K
HROWS = 80
DROWS = EP // 32 // 128
HALF = 5000
DUMP = HALF
SROWS = 5008
NB = 10
BR = 1024

_mesh = plsc.VectorSubcoreMesh(core_axis_name="c", subcore_axis_name="s")

_cp = pltpu.CompilerParams()
if "needs_layout_passes" in pltpu.CompilerParams.__dataclass_fields__:
    _cp = dataclasses.replace(_cp, needs_layout_passes=False)


def _deg_body(src_hbm, out_hbm, srcv, hist):
    c = lax.axis_index("c")
    s = lax.axis_index("s")
    w = c * NT + s

    pltpu.sync_copy(src_hbm.at[w], srcv)

    @pl.loop(0, HROWS)
    def _zero(r):
        for cc in range(8):
            hist[r, pl.ds(cc * 16, 16)] = jnp.zeros((16,), jnp.float32)

    ones = jnp.full((16,), 1.0, jnp.float32)

    @pl.loop(0, DROWS)
    def _rows(r):
        for cc in range(8):
            idx = srcv[r, pl.ds(cc * 16, 16)]
            plsc.addupdate_scatter(hist, [idx >> 7, idx & 127], ones)

    pltpu.sync_copy(hist, out_hbm.at[w])


@jax.jit
def _degree(src_d):
    k = pl.kernel(
        _deg_body,
        out_type=jax.ShapeDtypeStruct((32, HROWS, 128), jnp.float32),
        mesh=_mesh,
        compiler_params=_cp,
        scratch_types=[
            pltpu.VMEM((DROWS, 128), jnp.int32),
            pltpu.VMEM((HROWS, 128), jnp.float32),
        ],
    )
    return k(src_d)


def _proj_body(hist_ref, x_ref, w_ref, y_ref, dis_ref):
    deg = jnp.sum(hist_ref[...], axis=0)
    dis = jnp.where(deg > 0.0, lax.rsqrt(jnp.maximum(deg, 1.0)), 0.0)
    xp = jnp.dot(x_ref[...], w_ref[...], preferred_element_type=jnp.float32)
    y = xp * dis
    y_ref[0] = y[:, :CH]
    y_ref[1] = y[:, CH:]
    dis_ref[...] = dis


@jax.jit
def _project(hist, x, W):
    return pl.pallas_call(
        _proj_body,
        grid=(NB,),
        in_specs=[
            pl.BlockSpec((32, BR, 1), lambda i: (0, i, 0)),
            pl.BlockSpec((BR, C), lambda i: (i, 0)),
            pl.BlockSpec((C, C), lambda i: (0, 0)),
        ],
        out_specs=[
            pl.BlockSpec((2, BR, CH), lambda i: (0, i, 0)),
            pl.BlockSpec((BR, 1), lambda i: (i, 0)),
        ],
        out_shape=[
            jax.ShapeDtypeStruct((2, NP, CH), jnp.float32),
            jax.ShapeDtypeStruct((NP, 1), jnp.float32),
        ],
    )(hist, x, W)


def _segsum_body(y_hbm, src_hbm, dst_hbm, zeros_hbm, out_hbm,
                 srcv, dstv, buf0, buf1, slab, sem0, sem1):
    c = lax.axis_index("c")
    s = lax.axis_index("s")

    pltpu.sync_copy(src_hbm.at[s], srcv)
    pltpu.sync_copy(dst_hbm.at[s], dstv)
    pltpu.sync_copy(zeros_hbm.at[pl.ds(s * 312, 312)],
                    slab.at[pl.ds(s * 312, 312)])

    @pl.when(s == 0)
    def _ztail():
        pltpu.sync_copy(zeros_hbm.at[pl.ds(4992, 16)], slab.at[pl.ds(4992, 16)])

    plsc.subcore_barrier()

    def run(ci):
        yc = y_hbm.at[ci]
        pltpu.async_copy(yc.at[srcv.at[0]], buf0, sem0)

        @pl.loop(0, GCH, step=2)
        def _(j):
            pltpu.make_async_copy(yc.at[srcv.at[j]], buf0, sem0).wait()
            pltpu.async_copy(yc.at[srcv.at[j + 1]], buf1, sem1)
            pltpu.sync_copy(buf0, slab.at[dstv.at[j]], add=True)
            pltpu.make_async_copy(yc.at[srcv.at[j + 1]], buf1, sem1).wait()

            @pl.when(j < GCH - 2)
            def _pref():
                pltpu.async_copy(yc.at[srcv.at[j + 2]], buf0, sem0)

            pltpu.sync_copy(buf1, slab.at[dstv.at[j + 1]], add=True)

        plsc.subcore_barrier()
        pltpu.sync_copy(slab.at[pl.ds(s * 312, 312)],
                        out_hbm.at[ci, pl.ds(s * 312, 312)])

        @pl.when(s == 0)
        def _wtail():
            pltpu.sync_copy(slab.at[pl.ds(4992, 16)],
                            out_hbm.at[ci, pl.ds(4992, 16)])

    @pl.when(c == 0)
    def _c0():
        run(0)

    @pl.when(c == 1)
    def _c1():
        run(1)


@jax.jit
def _segsum(y, src_g, dst_g, zeros_slab):
    k = pl.kernel(
        _segsum_body,
        out_type=jax.ShapeDtypeStruct((2, SROWS, CH), jnp.float32),
        mesh=_mesh,
        scratch_types=[
            pltpu.VMEM((GCH, K), jnp.int32),
            pltpu.VMEM((GCH, K), jnp.int32),
            pltpu.VMEM((K, CH), jnp.float32),
            pltpu.VMEM((K, CH), jnp.float32),
            pltpu.VMEM_SHARED((SROWS, CH), jnp.float32),
            pltpu.SemaphoreType.DMA,
            pltpu.SemaphoreType.DMA,
        ],
    )
    return k(y, src_g, dst_g, zeros_slab)


def _fin_body(lo_ref, hi_ref, dis_ref, b_ref, out_ref):
    i = pl.program_id(0)
    dis = dis_ref[...]
    bias = b_ref[...]

    @pl.when(i < 5)
    def _lo():
        o = jnp.concatenate([lo_ref[0], lo_ref[1]], axis=1)
        out_ref[...] = o * dis + bias

    @pl.when(i >= 5)
    def _hi():
        o = jnp.concatenate([hi_ref[0], hi_ref[1]], axis=1)
        out_ref[...] = o * dis + bias


@jax.jit
def _finalize(out_lo, out_hi, dis, b2):
    return pl.pallas_call(
        _fin_body,
        grid=(NB,),
        in_specs=[
            pl.BlockSpec((2, 1000, CH), lambda i: (0, i % 5, 0)),
            pl.BlockSpec((2, 1000, CH), lambda i: (0, i % 5, 0)),
            pl.BlockSpec((1000, 1), lambda i: (i, 0)),
            pl.BlockSpec((1, C), lambda i: (0, 0)),
        ],
        out_specs=pl.BlockSpec((1000, C), lambda i: (i, 0)),
        out_shape=jax.ShapeDtypeStruct((N, C), jnp.float32),
    )(out_lo, out_hi, dis, b2)


def kernel(x, edge_index_K, W, b):
    edge = edge_index_K.astype(jnp.int32)
    src = edge[0]
    dst = edge[1]
    src_d = jnp.concatenate(
        [src, jnp.full((EP - E,), NP - 1, jnp.int32)]).reshape(32, DROWS, 128)
    pad = jnp.full((EP - E,), N, jnp.int32)
    src_g = jnp.concatenate([src, jnp.zeros((EP - E,), jnp.int32)]
                            ).reshape(NT, GCH, K)
    dst_p = jnp.concatenate([dst, pad])
    dst_lo = jnp.where(dst_p < HALF, dst_p, DUMP).reshape(NT, GCH, K)
    dst_hi = jnp.where(dst_p >= HALF, dst_p - HALF, DUMP).reshape(NT, GCH, K)
    x_pad = jnp.pad(x, ((0, NP - N), (0, 0)))
    zeros_slab = jnp.zeros((SROWS, CH), jnp.float32)

    hist = _degree(src_d).reshape(32, NP, 1)
    y, dis = _project(hist, x_pad, W)
    out_lo = _segsum(y, src_g, dst_lo, zeros_slab)
    out_hi = _segsum(y, src_g, dst_hi, zeros_slab)
    return _finalize(out_lo, out_hi, dis[:N], b.reshape(1, C))


# speedup vs baseline: 2.5428x; 2.5428x over previous
import dataclasses

import jax
import jax.numpy as jnp
from jax import lax
from jax.experimental import pallas as pl
from jax.experimental.pallas import tpu as pltpu
from jax.experimental.pallas import tpu_sc as plsc

N = 10000
NP = 10240
E = 160000
EP = 163840
C = 256
CH = 128
NT = 16
K = 100
GCH = (E // NT) // K
HROWS = 80
DROWS = EP // 32 // 128
HALF = 5000
DUMP = HALF
SROWS = 5008
NB = 10
BR = 1024

_mesh = plsc.VectorSubcoreMesh(core_axis_name="c", subcore_axis_name="s")

_cp = pltpu.CompilerParams()
if "needs_layout_passes" in pltpu.CompilerParams.__dataclass_fields__:
    _cp = dataclasses.replace(_cp, needs_layout_passes=False)


def _deg_body(src_hbm, out_hbm, srcv, hist):
    c = lax.axis_index("c")
    s = lax.axis_index("s")
    w = c * NT + s

    pltpu.sync_copy(src_hbm.at[w], srcv)

    @pl.loop(0, HROWS)
    def _zero(r):
        for cc in range(8):
            hist[r, pl.ds(cc * 16, 16)] = jnp.zeros((16,), jnp.float32)

    ones = jnp.full((16,), 1.0, jnp.float32)

    @pl.loop(0, DROWS)
    def _rows(r):
        for cc in range(8):
            idx = srcv[r, pl.ds(cc * 16, 16)]
            plsc.addupdate_scatter(hist, [idx >> 7, idx & 127], ones)

    pltpu.sync_copy(hist, out_hbm.at[w])


@jax.jit
def _degree(src_d):
    k = pl.kernel(
        _deg_body,
        out_type=jax.ShapeDtypeStruct((32, HROWS, 128), jnp.float32),
        mesh=_mesh,
        compiler_params=_cp,
        scratch_types=[
            pltpu.VMEM((DROWS, 128), jnp.int32),
            pltpu.VMEM((HROWS, 128), jnp.float32),
        ],
    )
    return k(src_d)


def _dis_body(hist_ref, dis_ref):
    deg = jnp.sum(hist_ref[...], axis=0)
    dis_ref[...] = jnp.where(deg > 0.0, lax.rsqrt(jnp.maximum(deg, 1.0)), 0.0)


@jax.jit
def _dis_grid(hist):
    return pl.pallas_call(
        _dis_body,
        out_shape=jax.ShapeDtypeStruct((HROWS, 128), jnp.float32),
    )(hist)


def _proj_body(dis_ref, x_ref, w_ref, y_ref):
    xp = jnp.dot(x_ref[...], w_ref[...], preferred_element_type=jnp.float32)
    y = xp * dis_ref[...]
    y_ref[0] = y[:, :CH]
    y_ref[1] = y[:, CH:]


@jax.jit
def _project(dis, x, W):
    return pl.pallas_call(
        _proj_body,
        grid=(NB,),
        in_specs=[
            pl.BlockSpec((BR, 1), lambda i: (i, 0)),
            pl.BlockSpec((BR, C), lambda i: (i, 0)),
            pl.BlockSpec((C, C), lambda i: (0, 0)),
        ],
        out_specs=pl.BlockSpec((2, BR, CH), lambda i: (0, i, 0)),
        out_shape=jax.ShapeDtypeStruct((2, NP, CH), jnp.float32),
    )(dis, x, W)


def _segsum_body(y_hbm, src_hbm, dst_hbm, zeros_hbm, out_hbm,
                 srcv, dstv, buf0, buf1, slab, sem0, sem1):
    c = lax.axis_index("c")
    s = lax.axis_index("s")

    pltpu.sync_copy(src_hbm.at[s], srcv)
    pltpu.sync_copy(dst_hbm.at[s], dstv)
    pltpu.sync_copy(zeros_hbm.at[pl.ds(s * 312, 312)],
                    slab.at[pl.ds(s * 312, 312)])

    @pl.when(s == 0)
    def _ztail():
        pltpu.sync_copy(zeros_hbm.at[pl.ds(4992, 16)], slab.at[pl.ds(4992, 16)])

    plsc.subcore_barrier()

    def run(ci):
        yc = y_hbm.at[ci]
        pltpu.async_copy(yc.at[srcv.at[0]], buf0, sem0)

        @pl.loop(0, GCH, step=2)
        def _(j):
            pltpu.make_async_copy(yc.at[srcv.at[j]], buf0, sem0).wait()
            pltpu.async_copy(yc.at[srcv.at[j + 1]], buf1, sem1)
            pltpu.sync_copy(buf0, slab.at[dstv.at[j]], add=True)
            pltpu.make_async_copy(yc.at[srcv.at[j + 1]], buf1, sem1).wait()

            @pl.when(j < GCH - 2)
            def _pref():
                pltpu.async_copy(yc.at[srcv.at[j + 2]], buf0, sem0)

            pltpu.sync_copy(buf1, slab.at[dstv.at[j + 1]], add=True)

        plsc.subcore_barrier()
        pltpu.sync_copy(slab.at[pl.ds(s * 312, 312)],
                        out_hbm.at[ci, pl.ds(s * 312, 312)])

        @pl.when(s == 0)
        def _wtail():
            pltpu.sync_copy(slab.at[pl.ds(4992, 16)],
                            out_hbm.at[ci, pl.ds(4992, 16)])

    @pl.when(c == 0)
    def _c0():
        run(0)

    @pl.when(c == 1)
    def _c1():
        run(1)


@jax.jit
def _segsum(y, src_g, dst_g, zeros_slab):
    k = pl.kernel(
        _segsum_body,
        out_type=jax.ShapeDtypeStruct((2, SROWS, CH), jnp.float32),
        mesh=_mesh,
        scratch_types=[
            pltpu.VMEM((GCH, K), jnp.int32),
            pltpu.VMEM((GCH, K), jnp.int32),
            pltpu.VMEM((K, CH), jnp.float32),
            pltpu.VMEM((K, CH), jnp.float32),
            pltpu.VMEM_SHARED((SROWS, CH), jnp.float32),
            pltpu.SemaphoreType.DMA,
            pltpu.SemaphoreType.DMA,
        ],
    )
    return k(y, src_g, dst_g, zeros_slab)


def _fin_body(lo_ref, hi_ref, dis_ref, b_ref, out_ref):
    i = pl.program_id(0)
    dis = dis_ref[...]
    bias = b_ref[...]

    @pl.when(i < 5)
    def _lo():
        o = jnp.concatenate([lo_ref[0], lo_ref[1]], axis=1)
        out_ref[...] = o * dis + bias

    @pl.when(i >= 5)
    def _hi():
        o = jnp.concatenate([hi_ref[0], hi_ref[1]], axis=1)
        out_ref[...] = o * dis + bias


@jax.jit
def _finalize(out_lo, out_hi, dis, b2):
    return pl.pallas_call(
        _fin_body,
        grid=(NB,),
        in_specs=[
            pl.BlockSpec((2, 1000, CH), lambda i: (0, i % 5, 0)),
            pl.BlockSpec((2, 1000, CH), lambda i: (0, i % 5, 0)),
            pl.BlockSpec((1000, 1), lambda i: (i, 0)),
            pl.BlockSpec((1, C), lambda i: (0, 0)),
        ],
        out_specs=pl.BlockSpec((1000, C), lambda i: (i, 0)),
        out_shape=jax.ShapeDtypeStruct((N, C), jnp.float32),
    )(out_lo, out_hi, dis, b2)


def kernel(x, edge_index_K, W, b):
    edge = edge_index_K.astype(jnp.int32)
    src = edge[0]
    dst = edge[1]
    src_d = jnp.concatenate(
        [src, jnp.full((EP - E,), NP - 1, jnp.int32)]).reshape(32, DROWS, 128)
    src_g = src.reshape(NT, GCH, K)
    dst_lo = jnp.where(dst < HALF, dst, DUMP).reshape(NT, GCH, K)
    dst_hi = jnp.where(dst >= HALF, dst - HALF, DUMP).reshape(NT, GCH, K)
    x_pad = jnp.pad(x, ((0, NP - N), (0, 0)))
    zeros_slab = jnp.zeros((SROWS, CH), jnp.float32)

    dis = _dis_grid(_degree(src_d)).reshape(NP, 1)
    y = _project(dis, x_pad, W)
    out_lo = _segsum(y, src_g, dst_lo, zeros_slab)
    out_hi = _segsum(y, src_g, dst_hi, zeros_slab)
    return _finalize(out_lo, out_hi, dis[:N], b.reshape(1, C))
